# baseline (speedup 1.0000x reference)
import functools

import jax
import jax.numpy as jnp
from jax.experimental import pallas as pl

_NUM_HEADS = 4
_SPARSE_RATIO = 0.0625
_NEG = -3.0e38


def _ctx_body(x_ref, o_ref):
    x = x_ref[...]
    avg = jnp.mean(x, axis=1)
    mx = jnp.max(x, axis=1)
    o_ref[...] = jnp.concatenate([avg, mx], axis=-1)


def _off_body(ctx_ref, w_ref, b_ref, o_ref):
    o_ref[...] = (
        jnp.dot(ctx_ref[...], w_ref[...], preferred_element_type=jnp.float32)
        + b_ref[...]
    )


def _main_body(x_ref, wpre_ref, bpre_ref, base_ref, off_ref, idx_ref, w_ref, *, k, inv_scale):
    x = x_ref[0]
    xp = jnp.dot(x, wpre_ref[...], preferred_element_type=jnp.float32) + bpre_ref[...]
    pro = base_ref[...] + off_ref[0]
    s = jax.lax.dot_general(
        xp, pro, (((1,), (1,)), ((), ())), preferred_element_type=jnp.float32
    ) * inv_scale
    nb, e = s.shape
    lane = jax.lax.broadcasted_iota(jnp.int32, (nb, e), 1)
    out_lane = jax.lax.broadcasted_iota(jnp.int32, (nb, k), 1)

    def step(j, carry):
        s, tv, ti = carry
        m = jnp.max(s, axis=1, keepdims=True)
        ei = jnp.min(jnp.where(s >= m, lane, e), axis=1, keepdims=True)
        hit = out_lane == j
        tv = jnp.where(hit, m, tv)
        ti = jnp.where(hit, ei, ti)
        s = jnp.where(lane == ei, _NEG, s)
        return s, tv, ti

    tv0 = jnp.zeros((nb, k), jnp.float32)
    ti0 = jnp.zeros((nb, k), jnp.int32)
    _, tv, ti = jax.lax.fori_loop(0, k, step, (s, tv0, ti0))
    ex = jnp.exp(tv - tv[:, :1])
    w = ex / jnp.sum(ex, axis=1, keepdims=True)
    idx_ref[0] = ti
    w_ref[0] = w


def kernel(X, prototype_base, W_ctx, b_ctx, W_pre, b_pre):
    B, N, D = X.shape
    E = prototype_base.shape[0]
    k = max(1, int(E * _SPARSE_RATIO))
    inv_scale = 1.0 / (float(_NUM_HEADS) * float(D // _NUM_HEADS) ** 0.5)

    ctx = pl.pallas_call(
        _ctx_body,
        out_shape=jax.ShapeDtypeStruct((B, 2 * D), jnp.float32),
        in_specs=[pl.BlockSpec((B, N, D), lambda: (0, 0, 0))],
        out_specs=pl.BlockSpec((B, 2 * D), lambda: (0, 0)),
    )(X)

    ec = 16
    cw = (E * D) // ec
    off2 = pl.pallas_call(
        _off_body,
        grid=(ec,),
        out_shape=jax.ShapeDtypeStruct((B, E * D), jnp.float32),
        in_specs=[
            pl.BlockSpec((B, 2 * D), lambda i: (0, 0)),
            pl.BlockSpec((2 * D, cw), lambda i: (0, i)),
            pl.BlockSpec((1, cw), lambda i: (0, i)),
        ],
        out_specs=pl.BlockSpec((B, cw), lambda i: (0, i)),
    )(ctx, W_ctx, b_ctx.reshape(1, E * D))
    off3 = off2.reshape(B, E, D)

    nb = 256
    grid = (B, N // nb)
    idx, w = pl.pallas_call(
        functools.partial(_main_body, k=k, inv_scale=inv_scale),
        grid=grid,
        out_shape=(
            jax.ShapeDtypeStruct((B, N, k), jnp.int32),
            jax.ShapeDtypeStruct((B, N, k), jnp.float32),
        ),
        in_specs=[
            pl.BlockSpec((1, nb, D), lambda b, n: (b, n, 0)),
            pl.BlockSpec((D, D), lambda b, n: (0, 0)),
            pl.BlockSpec((1, D), lambda b, n: (0, 0)),
            pl.BlockSpec((E, D), lambda b, n: (0, 0)),
            pl.BlockSpec((1, E, D), lambda b, n: (b, 0, 0)),
        ],
        out_specs=(
            pl.BlockSpec((1, nb, k), lambda b, n: (b, n, 0)),
            pl.BlockSpec((1, nb, k), lambda b, n: (b, n, 0)),
        ),
    )(X, W_pre, b_pre.reshape(1, D), prototype_base, off3)
    return (idx, w, jnp.asarray(E, dtype=jnp.int32))
